# Initial kernel scaffold; baseline (speedup 1.0000x reference)
import jax
import jax.numpy as jnp
from jax.experimental import pallas as pl


def kernel(gene_x, meth_x, mirna_x, gene_edge, cpg_edge, mirna_edge, gene_params, cpg_params, mirna_params):
    raise NotImplementedError("write your pallas kernel here")


# speedup vs baseline: 33.8597x; 33.8597x over previous
import functools

import jax
import jax.numpy as jnp
from jax import lax
from jax.experimental import pallas as pl
from jax.experimental.pallas import tpu as pltpu
from jax.experimental.pallas import tpu_sc as plsc

HID = 128
HEADS = 8
OUTC = HID // HEADS
NSUB = 16
C = 128
BN = 400

_f32 = jnp.float32


def _ln(t, g, b):
    mu = jnp.mean(t, axis=-1, keepdims=True)
    var = jnp.mean((t - mu) * (t - mu), axis=-1, keepdims=True)
    return (t - mu) * lax.rsqrt(var + 1e-5) * g + b


def _layer_tail(h, Wl, Wr, attf, G):
    xl = jnp.dot(h, Wl, preferred_element_type=_f32)
    xr = jnp.dot(h, Wr, preferred_element_type=_f32)
    s = xl + xr
    e = jnp.maximum(s, 0.0) + 0.2 * jnp.minimum(s, 0.0)
    logit8 = jnp.dot(e * attf, G, preferred_element_type=_f32)
    a = jnp.exp(logit8)
    ab = jnp.dot(a, G.T, preferred_element_type=_f32)
    return xl, xr, ab * xl, a


def _k1_body(x_ref, pW_ref, vec_ref, Wl_ref, Wr_ref, G_ref,
             h_ref, xl_ref, xr_ref, snum_ref, sden_ref):
    x = x_ref[...]
    t = jnp.dot(x, pW_ref[...], preferred_element_type=_f32) + vec_ref[0]
    h = jnp.maximum(_ln(t, vec_ref[1], vec_ref[2]), 0.0)
    xl, xr, snum, sden = _layer_tail(h, Wl_ref[...], Wr_ref[...], vec_ref[3],
                                     G_ref[...])
    h_ref[...] = h
    xl_ref[...] = xl
    xr_ref[...] = xr
    snum_ref[...] = snum
    sden_ref[...] = sden


def _merge(hprev, snum, sden, enum, eden, vec, G):
    num = snum + enum
    den = jnp.dot(sden + eden, G.T, preferred_element_type=_f32)
    agg = num / (den + 1e-16) + vec[0]
    el = jnp.where(agg > 0, agg, jnp.exp(jnp.minimum(agg, 0.0)) - 1.0)
    return _ln(el + hprev, vec[1], vec[2])


def _k2_body(h_ref, snum_ref, sden_ref, enum_ref, eden_ref, vec_ref,
             Wl_ref, Wr_ref, G_ref,
             h1_ref, xl_ref, xr_ref, snum2_ref, sden2_ref):
    G = G_ref[...]
    h1 = _merge(h_ref[...], snum_ref[...], sden_ref[...], enum_ref[...],
                eden_ref[...], vec_ref[...], G)
    xl, xr, snum, sden = _layer_tail(h1, Wl_ref[...], Wr_ref[...], vec_ref[3], G)
    h1_ref[...] = h1
    xl_ref[...] = xl
    xr_ref[...] = xr
    snum2_ref[...] = snum
    sden2_ref[...] = sden


def _k3_body(h_ref, snum_ref, sden_ref, enum_ref, eden_ref, vec_ref, G_ref,
             z_ref, *, inv_n):
    h2 = _merge(h_ref[...], snum_ref[...], sden_ref[...], enum_ref[...],
                eden_ref[...], vec_ref[...], G_ref[...])

    @pl.when(pl.program_id(0) == 0)
    def _():
        z_ref[...] = jnp.zeros_like(z_ref)

    z_ref[...] += jnp.sum(h2, axis=0, keepdims=True) * inv_n


def _row_spec(bn, cols):
    return pl.BlockSpec((bn, cols), lambda i: (i, 0))


def _full_spec(shape):
    nd = len(shape)
    return pl.BlockSpec(shape, lambda i: (0,) * nd)


@functools.partial(jax.jit, static_argnames=())
def _noop():
    pass


def _k1_call(xt, pW, vec, Wl, Wr, G):
    N = xt.shape[0]
    grid = (N // BN,)
    out = (
        jax.ShapeDtypeStruct((N, HID), _f32),
        jax.ShapeDtypeStruct((N, HID), _f32),
        jax.ShapeDtypeStruct((N, HID), _f32),
        jax.ShapeDtypeStruct((N, HID), _f32),
        jax.ShapeDtypeStruct((N, HEADS), _f32),
    )
    return pl.pallas_call(
        _k1_body,
        grid=grid,
        in_specs=[_row_spec(BN, HID), _full_spec((HID, HID)),
                  _full_spec((4, HID)), _full_spec((HID, HID)),
                  _full_spec((HID, HID)), _full_spec((HID, HEADS))],
        out_specs=(_row_spec(BN, HID), _row_spec(BN, HID), _row_spec(BN, HID),
                   _row_spec(BN, HID), _row_spec(BN, HEADS)),
        out_shape=out,
    )(xt, pW, vec, Wl, Wr, G)


def _k2_call(h, snum, sden, enum, eden, vec, Wl, Wr, G):
    N = h.shape[0]
    grid = (N // BN,)
    out = (
        jax.ShapeDtypeStruct((N, HID), _f32),
        jax.ShapeDtypeStruct((N, HID), _f32),
        jax.ShapeDtypeStruct((N, HID), _f32),
        jax.ShapeDtypeStruct((N, HID), _f32),
        jax.ShapeDtypeStruct((N, HEADS), _f32),
    )
    return pl.pallas_call(
        _k2_body,
        grid=grid,
        in_specs=[_row_spec(BN, HID), _row_spec(BN, HID), _row_spec(BN, HEADS),
                  _row_spec(BN, HID), _row_spec(BN, HEADS),
                  _full_spec((4, HID)), _full_spec((HID, HID)),
                  _full_spec((HID, HID)), _full_spec((HID, HEADS))],
        out_specs=(_row_spec(BN, HID), _row_spec(BN, HID), _row_spec(BN, HID),
                   _row_spec(BN, HID), _row_spec(BN, HEADS)),
        out_shape=out,
    )(h, snum, sden, enum, eden, vec, Wl, Wr, G)


def _k3_call(h, snum, sden, enum, eden, vec, G):
    N = h.shape[0]
    grid = (N // BN,)
    return pl.pallas_call(
        functools.partial(_k3_body, inv_n=1.0 / N),
        grid=grid,
        in_specs=[_row_spec(BN, HID), _row_spec(BN, HID), _row_spec(BN, HEADS),
                  _row_spec(BN, HID), _row_spec(BN, HEADS),
                  _full_spec((4, HID)), _full_spec((HID, HEADS))],
        out_specs=pl.BlockSpec((1, HID), lambda i: (0, 0)),
        out_shape=jax.ShapeDtypeStruct((1, HID), _f32),
        compiler_params=pltpu.CompilerParams(
            dimension_semantics=("arbitrary",)),
    )(h, snum, sden, enum, eden, vec, G)


@functools.lru_cache(maxsize=None)
def _make_sc_edge(N_pad, ntc):
    TROWS = N_pad // NSUB
    HPC = HEADS // 2
    mesh = plsc.VectorSubcoreMesh(core_axis_name="c", subcore_axis_name="s")

    @functools.partial(
        pl.kernel,
        out_type=(jax.ShapeDtypeStruct((HEADS, N_pad, OUTC), _f32),
                  jax.ShapeDtypeStruct((HEADS * N_pad,), _f32)),
        mesh=mesh,
        compiler_params=pltpu.CompilerParams(use_tc_tiling_on_sc=False),
        scratch_types=[
            pltpu.VMEM((ntc, C), jnp.int32),
            pltpu.VMEM((ntc, C), jnp.int32),
            pltpu.VMEM((HEADS, OUTC), _f32),
            pltpu.VMEM((C,), jnp.int32),
            pltpu.VMEM((C, OUTC), _f32),
            pltpu.VMEM((C, OUTC), _f32),
            pltpu.VMEM((C, OUTC), _f32),
            pltpu.VMEM((C,), _f32),
            pltpu.VMEM((128, OUTC), _f32),
            pltpu.VMEM((1024,), _f32),
            pltpu.VMEM_SHARED((N_pad + 8, OUTC), _f32),
            pltpu.VMEM_SHARED((N_pad + 8,), _f32),
        ],
    )
    def sc_edge(xl2, xr2, src3, dst3, att, enum_out, eden_out,
                srcv, dstv, attv, idxb, xlb, xrb, wb, ab, zb, zb1,
                num_s, den_s):
        cid = lax.axis_index("c")
        sid = lax.axis_index("s")

        pltpu.sync_copy(src3.at[sid], srcv)
        pltpu.sync_copy(dst3.at[sid], dstv)
        pltpu.sync_copy(att, attv)

        zero16 = jnp.zeros((OUTC,), _f32)
        iot = lax.iota(jnp.int32, OUTC)
        perms = [jnp.bitwise_xor(iot, 1 << k) for k in range(4)]

        def _zb_zero(i, _):
            zb[i, :] = zero16
            return 0
        lax.fori_loop(0, 128, _zb_zero, 0)

        def _zb1_zero(i, _):
            zb1[pl.ds(i * OUTC, OUTC)] = zero16
            return 0
        lax.fori_loop(0, 1024 // OUTC, _zb1_zero, 0)

        my_base = sid * TROWS

        for k in range(HPC):
            hh = cid * HPC + k
            atth = attv[hh, :]

            nfull, tail = divmod(TROWS, 128)
            for j in range(nfull):
                pltpu.sync_copy(zb, num_s.at[pl.ds(my_base + j * 128, 128)])
            if tail:
                pltpu.sync_copy(zb.at[pl.ds(0, tail)],
                                num_s.at[pl.ds(my_base + nfull * 128, tail)])
            nfull1, tail1 = divmod(TROWS, 1024)
            for j in range(nfull1):
                pltpu.sync_copy(zb1, den_s.at[pl.ds(my_base + j * 1024, 1024)])
            if tail1:
                pltpu.sync_copy(zb1.at[pl.ds(0, tail1)],
                                den_s.at[pl.ds(my_base + nfull1 * 1024, tail1)])

            plsc.subcore_barrier()

            def _chunk(ci, _):
                def _fill_src(jj, _):
                    v = srcv[ci, pl.ds(jj * OUTC, OUTC)]
                    idxb[pl.ds(jj * OUTC, OUTC)] = v * HEADS + hh
                    return 0
                lax.fori_loop(0, C // OUTC, _fill_src, 0)
                pltpu.sync_copy(xl2.at[idxb], xlb)

                def _fill_dst(jj, _):
                    v = dstv[ci, pl.ds(jj * OUTC, OUTC)]
                    idxb[pl.ds(jj * OUTC, OUTC)] = v * HEADS + hh
                    return 0
                lax.fori_loop(0, C // OUTC, _fill_dst, 0)
                pltpu.sync_copy(xr2.at[idxb], xrb)

                def _group(g, _):
                    acc = jnp.zeros((OUTC,), _f32)
                    for j in range(OUTC):
                        i = g * OUTC + j
                        xlv = xlb[i, :]
                        xrv = xrb[i, :]
                        s = xlv + xrv
                        e = jnp.maximum(s, 0.0) + 0.2 * jnp.minimum(s, 0.0)
                        red = e * atth
                        for pm in perms:
                            red = red + red.at[pm].get(
                                mode="promise_in_bounds")
                        av = jnp.exp(red)
                        wb[i, :] = av * xlv
                        acc = jnp.where(iot == j, av, acc)
                    ab[pl.ds(g * OUTC, OUTC)] = acc
                    return 0
                lax.fori_loop(0, C // OUTC, _group, 0)

                pltpu.sync_copy(wb, num_s.at[dstv.at[ci]], add=True)
                pltpu.sync_copy(ab, den_s.at[dstv.at[ci]], add=True)
                return 0
            lax.fori_loop(0, ntc, _chunk, 0)

            plsc.subcore_barrier()

            pltpu.sync_copy(num_s.at[pl.ds(my_base, TROWS)],
                            enum_out.at[hh, pl.ds(my_base, TROWS)])
            pltpu.sync_copy(den_s.at[pl.ds(my_base, TROWS)],
                            eden_out.at[pl.ds(hh * N_pad + my_base, TROWS)])

    return sc_edge


def _sc_edge_call(N, xl, xr, edge, att):
    E = edge.shape[1]
    N_pad = -(-N // (NSUB * 8)) * (NSUB * 8)
    ntc = -(-E // (NSUB * C))
    E_pad = NSUB * C * ntc
    src = jnp.concatenate([edge[0], jnp.zeros((E_pad - E,), jnp.int32)])
    dst = jnp.concatenate([edge[1], jnp.full((E_pad - E,), N_pad, jnp.int32)])
    src3 = src.reshape(NSUB, ntc, C)
    dst3 = dst.reshape(NSUB, ntc, C)
    xl2 = xl.reshape(N * HEADS, OUTC)
    xr2 = xr.reshape(N * HEADS, OUTC)
    enum, eden = _make_sc_edge(N_pad, ntc)(xl2, xr2, src3, dst3, att)
    enum_t = jnp.transpose(enum, (1, 0, 2))[:N].reshape(N, HID)
    eden_t = jnp.transpose(eden.reshape(HEADS, N_pad), (1, 0))[:N]
    return enum_t, eden_t


def _encode(x, edge, p):
    N = x.shape[1]
    xt = jnp.transpose(x)
    G = (jnp.arange(HID, dtype=jnp.int32)[:, None] // OUTC ==
         jnp.arange(HEADS, dtype=jnp.int32)[None, :]).astype(_f32)

    l0, l1 = p['layers'][0], p['layers'][1]
    vec1 = jnp.stack([p['pb'], p['pg'], p['pB'], l0['att'].reshape(HID)])
    vec2 = jnp.stack([l0['bias'], l0['g'], l0['b'], l1['att'].reshape(HID)])
    vec3 = jnp.stack([l1['bias'], l1['g'], l1['b'], jnp.zeros((HID,), _f32)])

    h0, xl, xr, snum, sden = _k1_call(xt, p['pW'], vec1, l0['Wl'], l0['Wr'], G)
    enum, eden = _sc_edge_call(N, xl, xr, edge, l0['att'])
    h1, xl2, xr2, snum2, sden2 = _k2_call(h0, snum, sden, enum, eden, vec2,
                                          l1['Wl'], l1['Wr'], G)
    enum2, eden2 = _sc_edge_call(N, xl2, xr2, edge, l1['att'])
    z = _k3_call(h1, snum2, sden2, enum2, eden2, vec3, G)
    return z


def kernel(gene_x, meth_x, mirna_x, gene_edge, cpg_edge, mirna_edge,
           gene_params, cpg_params, mirna_params):
    B = gene_x.shape[0]
    zg = _encode(gene_x, gene_edge, gene_params)
    zc = _encode(meth_x, cpg_edge, cpg_params)
    zm = _encode(mirna_x, mirna_edge, mirna_params)
    return (jnp.broadcast_to(zg, (B, HID)),
            jnp.broadcast_to(zc, (B, HID)),
            jnp.broadcast_to(zm, (B, HID)))


# speedup vs baseline: 66.5169x; 1.9645x over previous
import functools

import jax
import jax.numpy as jnp
from jax import lax
from jax.experimental import pallas as pl
from jax.experimental.pallas import tpu as pltpu
from jax.experimental.pallas import tpu_sc as plsc

HID = 128
HEADS = 8
OUTC = HID // HEADS
NSUB = 16
C = 128
BN = 400

_f32 = jnp.float32


def _ln(t, g, b):
    mu = jnp.mean(t, axis=-1, keepdims=True)
    var = jnp.mean((t - mu) * (t - mu), axis=-1, keepdims=True)
    return (t - mu) * lax.rsqrt(var + 1e-5) * g + b


def _layer_tail(h, Wl, Wr, attf, G):
    xl = jnp.dot(h, Wl, preferred_element_type=_f32)
    xr = jnp.dot(h, Wr, preferred_element_type=_f32)
    s = xl + xr
    e = jnp.maximum(s, 0.0) + 0.2 * jnp.minimum(s, 0.0)
    logit8 = jnp.dot(e * attf, G, preferred_element_type=_f32)
    a = jnp.exp(logit8)
    ab = jnp.dot(a, G.T, preferred_element_type=_f32)
    return xl, xr, ab * xl, a


def _k1_body(x_ref, pW_ref, vec_ref, Wl_ref, Wr_ref, G_ref,
             h_ref, xl_ref, xr_ref, snum_ref, sden_ref):
    x = x_ref[...]
    t = jnp.dot(x, pW_ref[...], preferred_element_type=_f32) + vec_ref[0]
    h = jnp.maximum(_ln(t, vec_ref[1], vec_ref[2]), 0.0)
    xl, xr, snum, sden = _layer_tail(h, Wl_ref[...], Wr_ref[...], vec_ref[3],
                                     G_ref[...])
    h_ref[...] = h
    xl_ref[...] = xl
    xr_ref[...] = xr
    snum_ref[...] = snum
    sden_ref[...] = sden


def _merge(hprev, snum, sden, enum, eden, vec, G):
    num = snum + enum
    den = jnp.dot(sden + eden, G.T, preferred_element_type=_f32)
    agg = num / (den + 1e-16) + vec[0]
    el = jnp.where(agg > 0, agg, jnp.exp(jnp.minimum(agg, 0.0)) - 1.0)
    return _ln(el + hprev, vec[1], vec[2])


def _k2_body(h_ref, snum_ref, sden_ref, enum_ref, eden_ref, vec_ref,
             Wl_ref, Wr_ref, G_ref,
             h1_ref, xl_ref, xr_ref, snum2_ref, sden2_ref):
    G = G_ref[...]
    h1 = _merge(h_ref[...], snum_ref[...], sden_ref[...], enum_ref[...],
                eden_ref[...], vec_ref[...], G)
    xl, xr, snum, sden = _layer_tail(h1, Wl_ref[...], Wr_ref[...], vec_ref[3], G)
    h1_ref[...] = h1
    xl_ref[...] = xl
    xr_ref[...] = xr
    snum2_ref[...] = snum
    sden2_ref[...] = sden


def _k3_body(h_ref, snum_ref, sden_ref, enum_ref, eden_ref, vec_ref, G_ref,
             z_ref, *, inv_n):
    h2 = _merge(h_ref[...], snum_ref[...], sden_ref[...], enum_ref[...],
                eden_ref[...], vec_ref[...], G_ref[...])

    @pl.when(pl.program_id(0) == 0)
    def _():
        z_ref[...] = jnp.zeros_like(z_ref)

    z_ref[...] += jnp.sum(h2, axis=0, keepdims=True) * inv_n


def _row_spec(bn, cols):
    return pl.BlockSpec((bn, cols), lambda i: (i, 0))


def _full_spec(shape):
    nd = len(shape)
    return pl.BlockSpec(shape, lambda i: (0,) * nd)


@functools.partial(jax.jit, static_argnames=())
def _noop():
    pass


def _k1_call(xt, pW, vec, Wl, Wr, G):
    N = xt.shape[0]
    grid = (N // BN,)
    out = (
        jax.ShapeDtypeStruct((N, HID), _f32),
        jax.ShapeDtypeStruct((N, HID), _f32),
        jax.ShapeDtypeStruct((N, HID), _f32),
        jax.ShapeDtypeStruct((N, HID), _f32),
        jax.ShapeDtypeStruct((N, HEADS), _f32),
    )
    return pl.pallas_call(
        _k1_body,
        grid=grid,
        in_specs=[_row_spec(BN, HID), _full_spec((HID, HID)),
                  _full_spec((4, HID)), _full_spec((HID, HID)),
                  _full_spec((HID, HID)), _full_spec((HID, HEADS))],
        out_specs=(_row_spec(BN, HID), _row_spec(BN, HID), _row_spec(BN, HID),
                   _row_spec(BN, HID), _row_spec(BN, HEADS)),
        out_shape=out,
    )(xt, pW, vec, Wl, Wr, G)


def _k2_call(h, snum, sden, enum, eden, vec, Wl, Wr, G):
    N = h.shape[0]
    grid = (N // BN,)
    out = (
        jax.ShapeDtypeStruct((N, HID), _f32),
        jax.ShapeDtypeStruct((N, HID), _f32),
        jax.ShapeDtypeStruct((N, HID), _f32),
        jax.ShapeDtypeStruct((N, HID), _f32),
        jax.ShapeDtypeStruct((N, HEADS), _f32),
    )
    return pl.pallas_call(
        _k2_body,
        grid=grid,
        in_specs=[_row_spec(BN, HID), _row_spec(BN, HID), _row_spec(BN, HEADS),
                  _row_spec(BN, HID), _row_spec(BN, HEADS),
                  _full_spec((4, HID)), _full_spec((HID, HID)),
                  _full_spec((HID, HID)), _full_spec((HID, HEADS))],
        out_specs=(_row_spec(BN, HID), _row_spec(BN, HID), _row_spec(BN, HID),
                   _row_spec(BN, HID), _row_spec(BN, HEADS)),
        out_shape=out,
    )(h, snum, sden, enum, eden, vec, Wl, Wr, G)


def _k3_call(h, snum, sden, enum, eden, vec, G):
    N = h.shape[0]
    grid = (N // BN,)
    return pl.pallas_call(
        functools.partial(_k3_body, inv_n=1.0 / N),
        grid=grid,
        in_specs=[_row_spec(BN, HID), _row_spec(BN, HID), _row_spec(BN, HEADS),
                  _row_spec(BN, HID), _row_spec(BN, HEADS),
                  _full_spec((4, HID)), _full_spec((HID, HEADS))],
        out_specs=pl.BlockSpec((1, HID), lambda i: (0, 0)),
        out_shape=jax.ShapeDtypeStruct((1, HID), _f32),
        compiler_params=pltpu.CompilerParams(
            dimension_semantics=("arbitrary",)),
    )(h, snum, sden, enum, eden, vec, G)


NBUF = 3


@functools.lru_cache(maxsize=None)
def _make_sc_edge(N_pad, ntc):
    TROWS = N_pad // NSUB
    HPC = HEADS // 2
    nsteps = ntc // NBUF
    mesh = plsc.VectorSubcoreMesh(core_axis_name="c", subcore_axis_name="s")

    @functools.partial(
        pl.kernel,
        out_type=(jax.ShapeDtypeStruct((HEADS, N_pad, OUTC), _f32),
                  jax.ShapeDtypeStruct((HEADS * N_pad,), _f32)),
        mesh=mesh,
        compiler_params=pltpu.CompilerParams(use_tc_tiling_on_sc=False),
        scratch_types=[
            pltpu.VMEM((ntc, C), jnp.int32),
            pltpu.VMEM((ntc, C), jnp.int32),
            pltpu.VMEM((HEADS, OUTC), _f32),
            pltpu.VMEM((NBUF, C), jnp.int32),
            pltpu.VMEM((NBUF, C), jnp.int32),
            pltpu.VMEM((NBUF, C, OUTC), _f32),
            pltpu.VMEM((NBUF, C, OUTC), _f32),
            pltpu.VMEM((NBUF, C, OUTC), _f32),
            pltpu.VMEM((NBUF, C), _f32),
            pltpu.VMEM((64, OUTC), _f32),
            pltpu.VMEM((512,), _f32),
            pltpu.VMEM_SHARED((N_pad + 8, OUTC), _f32),
            pltpu.VMEM_SHARED((N_pad + 8,), _f32),
            pltpu.SemaphoreType.DMA((NBUF,)),
            pltpu.SemaphoreType.DMA((NBUF,)),
        ],
    )
    def sc_edge(xl2, xr2, src3, dst3, att, enum_out, eden_out,
                srcv, dstv, attv, idxl, idxr, xlb, xrb, wb, ab, zb, zb1,
                num_s, den_s, semg, sems):
        cid = lax.axis_index("c")
        sid = lax.axis_index("s")

        pltpu.sync_copy(src3.at[sid], srcv)
        pltpu.sync_copy(dst3.at[sid], dstv)
        pltpu.sync_copy(att, attv)

        zero16 = jnp.zeros((OUTC,), _f32)
        iot = lax.iota(jnp.int32, OUTC)
        perms = [jnp.bitwise_xor(iot, 1 << k) for k in range(4)]

        def _zb_zero(i, _):
            zb[i, :] = zero16
            return 0
        lax.fori_loop(0, 64, _zb_zero, 0)

        def _zb1_zero(i, _):
            zb1[pl.ds(i * OUTC, OUTC)] = zero16
            return 0
        lax.fori_loop(0, 512 // OUTC, _zb1_zero, 0)

        my_base = sid * TROWS

        def _head(k, _):
            hh = cid * HPC + k
            atth = attv[hh, :]

            nfull, tail = divmod(TROWS, 64)
            for j in range(nfull):
                pltpu.sync_copy(zb, num_s.at[pl.ds(my_base + j * 64, 64)])
            if tail:
                pltpu.sync_copy(zb.at[pl.ds(0, tail)],
                                num_s.at[pl.ds(my_base + nfull * 64, tail)])
            nfull1, tail1 = divmod(TROWS, 512)
            for j in range(nfull1):
                pltpu.sync_copy(zb1, den_s.at[pl.ds(my_base + j * 512, 512)])
            if tail1:
                pltpu.sync_copy(zb1.at[pl.ds(0, tail1)],
                                den_s.at[pl.ds(my_base + nfull1 * 512, tail1)])

            plsc.subcore_barrier()

            def _fill_fire(b, ci):
                for j in range(C // OUTC):
                    v = srcv[ci, pl.ds(j * OUTC, OUTC)]
                    idxl[b, pl.ds(j * OUTC, OUTC)] = v * HEADS + hh
                    w = dstv[ci, pl.ds(j * OUTC, OUTC)]
                    idxr[b, pl.ds(j * OUTC, OUTC)] = w * HEADS + hh
                pltpu.async_copy(xl2.at[idxl.at[b]], xlb.at[b], semg.at[b])
                pltpu.async_copy(xr2.at[idxr.at[b]], xrb.at[b], semg.at[b])

            def _wait_gather(b):
                pltpu.make_async_copy(xl2.at[idxl.at[b]], xlb.at[b],
                                      semg.at[b]).wait()
                pltpu.make_async_copy(xr2.at[idxr.at[b]], xrb.at[b],
                                      semg.at[b]).wait()

            def _wait_scatter(b):
                pltpu.make_async_copy(wb.at[b], num_s.at[dstv.at[0]],
                                      sems.at[b]).wait()
                pltpu.make_async_copy(ab.at[b], den_s.at[dstv.at[0]],
                                      sems.at[b]).wait()

            for b in range(NBUF):
                _fill_fire(b, b)

            def _step(s, _):
                for b in range(NBUF):
                    ci = s * NBUF + b
                    _wait_gather(b)

                    @pl.when(s > 0)
                    def _():
                        _wait_scatter(b)

                    def _group(g, _):
                        acc = jnp.zeros((OUTC,), _f32)
                        for j in range(OUTC):
                            i = g * OUTC + j
                            xlv = xlb[b, i, :]
                            xrv = xrb[b, i, :]
                            sv = xlv + xrv
                            e = (jnp.maximum(sv, 0.0)
                                 + 0.2 * jnp.minimum(sv, 0.0))
                            red = e * atth
                            for pm in perms:
                                red = red + red.at[pm].get(
                                    mode="promise_in_bounds")
                            av = jnp.exp(red)
                            wb[b, i, :] = av * xlv
                            acc = jnp.where(iot == j, av, acc)
                        ab[b, pl.ds(g * OUTC, OUTC)] = acc
                        return 0
                    lax.fori_loop(0, C // OUTC, _group, 0)

                    pltpu.async_copy(wb.at[b], num_s.at[dstv.at[ci]],
                                     sems.at[b], add=True)
                    pltpu.async_copy(ab.at[b], den_s.at[dstv.at[ci]],
                                     sems.at[b], add=True)

                    @pl.when(ci + NBUF < ntc)
                    def _():
                        _fill_fire(b, ci + NBUF)
                return 0
            lax.fori_loop(0, nsteps, _step, 0)

            for b in range(NBUF):
                _wait_scatter(b)

            plsc.subcore_barrier()

            pltpu.sync_copy(num_s.at[pl.ds(my_base, TROWS)],
                            enum_out.at[hh, pl.ds(my_base, TROWS)])
            pltpu.sync_copy(den_s.at[pl.ds(my_base, TROWS)],
                            eden_out.at[pl.ds(hh * N_pad + my_base, TROWS)])
            return 0

        lax.fori_loop(0, HPC, _head, 0)

    return sc_edge


def _sc_edge_call(N, xl, xr, edge, att):
    E = edge.shape[1]
    N_pad = -(-N // (NSUB * 8)) * (NSUB * 8)
    ntc = -(-E // (NSUB * C))
    ntc = -(-ntc // NBUF) * NBUF
    E_pad = NSUB * C * ntc
    src = jnp.concatenate([edge[0], jnp.zeros((E_pad - E,), jnp.int32)])
    dst = jnp.concatenate([edge[1], jnp.full((E_pad - E,), N_pad, jnp.int32)])
    src3 = src.reshape(NSUB, ntc, C)
    dst3 = dst.reshape(NSUB, ntc, C)
    xl2 = xl.reshape(N * HEADS, OUTC)
    xr2 = xr.reshape(N * HEADS, OUTC)
    enum, eden = _make_sc_edge(N_pad, ntc)(xl2, xr2, src3, dst3, att)
    enum_t = jnp.transpose(enum, (1, 0, 2))[:N].reshape(N, HID)
    eden_t = jnp.transpose(eden.reshape(HEADS, N_pad), (1, 0))[:N]
    return enum_t, eden_t


def _encode(x, edge, p):
    N = x.shape[1]
    xt = jnp.transpose(x)
    G = (jnp.arange(HID, dtype=jnp.int32)[:, None] // OUTC ==
         jnp.arange(HEADS, dtype=jnp.int32)[None, :]).astype(_f32)

    l0, l1 = p['layers'][0], p['layers'][1]
    vec1 = jnp.stack([p['pb'], p['pg'], p['pB'], l0['att'].reshape(HID)])
    vec2 = jnp.stack([l0['bias'], l0['g'], l0['b'], l1['att'].reshape(HID)])
    vec3 = jnp.stack([l1['bias'], l1['g'], l1['b'], jnp.zeros((HID,), _f32)])

    h0, xl, xr, snum, sden = _k1_call(xt, p['pW'], vec1, l0['Wl'], l0['Wr'], G)
    enum, eden = _sc_edge_call(N, xl, xr, edge, l0['att'])
    h1, xl2, xr2, snum2, sden2 = _k2_call(h0, snum, sden, enum, eden, vec2,
                                          l1['Wl'], l1['Wr'], G)
    enum2, eden2 = _sc_edge_call(N, xl2, xr2, edge, l1['att'])
    z = _k3_call(h1, snum2, sden2, enum2, eden2, vec3, G)
    return z


def kernel(gene_x, meth_x, mirna_x, gene_edge, cpg_edge, mirna_edge,
           gene_params, cpg_params, mirna_params):
    B = gene_x.shape[0]
    zg = _encode(gene_x, gene_edge, gene_params)
    zc = _encode(meth_x, cpg_edge, cpg_params)
    zm = _encode(mirna_x, mirna_edge, mirna_params)
    return (jnp.broadcast_to(zg, (B, HID)),
            jnp.broadcast_to(zc, (B, HID)),
            jnp.broadcast_to(zm, (B, HID)))
